# baseline (speedup 1.0000x reference)
import jax
import jax.numpy as jnp
from jax.experimental import pallas as pl

_B = 8192
_K = 8192
_BM = 256


def _body(p_ref, a_ref, vote_ref, idx_ref):
    p = p_ref[...]
    a = a_ref[...]
    vote = jax.lax.dot_general(
        p, a, (((1,), (0,)), ((), ())),
        preferred_element_type=jnp.float32,
        precision=jax.lax.Precision.HIGHEST)
    m = jnp.max(p, axis=1, keepdims=True)
    col = jax.lax.broadcasted_iota(jnp.int32, p.shape, 1)
    idx = jnp.min(jnp.where(p == m, col, _K), axis=1)
    vote_ref[...] = vote[:, 0]
    idx_ref[...] = idx


def kernel(probabilities, approximations, soft):
    a2d = approximations.reshape(_K, 1)
    grid = (_B // _BM,)
    vote, idx = pl.pallas_call(
        _body,
        grid=grid,
        in_specs=[
            pl.BlockSpec((_BM, _K), lambda i: (i, 0)),
            pl.BlockSpec((_K, 1), lambda i: (0, 0)),
        ],
        out_specs=[
            pl.BlockSpec((_BM,), lambda i: (i,)),
            pl.BlockSpec((_BM,), lambda i: (i,)),
        ],
        out_shape=[
            jax.ShapeDtypeStruct((_B,), jnp.float32),
            jax.ShapeDtypeStruct((_B,), jnp.int32),
        ],
    )(probabilities, a2d)
    winner_preds = approximations[idx]
    return jnp.where(soft, vote, winner_preds)


# speedup vs baseline: 2.5079x; 2.5079x over previous
import jax
import jax.numpy as jnp
from jax.experimental import pallas as pl

_B = 8192
_K = 8192
_BM = 256


def _body(p_ref, a_ref, vote_ref, idx_ref):
    p = p_ref[...]
    a = a_ref[...]
    vote = jnp.sum(p * a, axis=1)
    m = jnp.max(p, axis=1, keepdims=True)
    col = jax.lax.broadcasted_iota(jnp.int32, p.shape, 1)
    idx = jnp.min(jnp.where(p == m, col, _K), axis=1)
    vote_ref[...] = vote
    idx_ref[...] = idx


def kernel(probabilities, approximations, soft):
    a2d = approximations.reshape(1, _K)
    grid = (_B // _BM,)
    vote, idx = pl.pallas_call(
        _body,
        grid=grid,
        in_specs=[
            pl.BlockSpec((_BM, _K), lambda i: (i, 0)),
            pl.BlockSpec((1, _K), lambda i: (0, 0)),
        ],
        out_specs=[
            pl.BlockSpec((_BM,), lambda i: (i,)),
            pl.BlockSpec((_BM,), lambda i: (i,)),
        ],
        out_shape=[
            jax.ShapeDtypeStruct((_B,), jnp.float32),
            jax.ShapeDtypeStruct((_B,), jnp.int32),
        ],
    )(probabilities, a2d)
    winner_preds = approximations[idx]
    return jnp.where(soft, vote, winner_preds)
